# baseline (speedup 1.0000x reference)
import functools

import jax
import jax.numpy as jnp
from jax import lax
from jax.experimental import pallas as pl
from jax.experimental.pallas import tpu as pltpu
from jax.experimental.pallas import tpu_sc as plsc

N = 100000
K = 128
CH = 160
CHK = CH * K
NCH = N // CH
NW = 32
GROUPS = CH // 16
MAXC = -(-NCH // NW)
SLICE = CHK // 16

_mesh = plsc.VectorSubcoreMesh(core_axis_name="c", subcore_axis_name="s")


@functools.partial(
    pl.kernel,
    mesh=_mesh,
    out_type=jax.ShapeDtypeStruct((N * K,), jnp.float32),
    scratch_types=(
        [pltpu.VMEM((CHK,), jnp.float32) for _ in range(2)]
        + [pltpu.VMEM((CH,), jnp.int32) for _ in range(2)]
        + [pltpu.VMEM((CH,), jnp.int32) for _ in range(2)]
        + [pltpu.VMEM((CH,), jnp.int32) for _ in range(2)]
        + [pltpu.VMEM((128,), jnp.int32) for _ in range(2)]
        + [pltpu.VMEM((32,), jnp.int32) for _ in range(2)]
        + [pltpu.VMEM((128,), jnp.float32),
           pltpu.VMEM((32,), jnp.float32)]
        + [pltpu.VMEM_SHARED((CHK,), jnp.float32)]
        + [pltpu.SemaphoreType.DMA for _ in range(10)]
    ),
    compiler_params=pltpu.CompilerParams(needs_layout_passes=False),
)
def _one_hot_sc(atom_hbm, out_hbm, buf0, buf1, idx0, idx1, fi0, fi1,
                sx0, sx1, fa0, fa1, fb0, fb1, onesA, onesB, zsh,
                os0, os1, is0, is1, zs0, zs1, ss0, ss1, sis0, sis1):
    bufs = (buf0, buf1)
    idxs = (idx0, idx1)
    fis = (fi0, fi1)
    sidx = (sx0, sx1)
    fas = (fa0, fa1)
    fbs = (fb0, fb1)
    outsems = (os0, os1)
    idxsems = (is0, is1)
    zsems = (zs0, zs1)
    ssems = (ss0, ss1)
    sidxsems = (sis0, sis1)

    info = plsc.get_sparse_core_info()
    sid = lax.axis_index("s")
    wid = sid * info.num_cores + lax.axis_index("c")

    zvec = jnp.zeros((16,), jnp.float32)
    ovec = jnp.ones((16,), jnp.float32)
    row_off = lax.iota(jnp.int32, 16) * K

    n_mine = (NCH - wid + NW - 1) // NW

    def _prefetch_idx(m, ref, sem):
        chunk = wid + m * NW
        pltpu.async_copy(atom_hbm.at[pl.ds(chunk * CH, CH)], ref, sem)

    def _fill(b):
        buf, idx_v, fi = bufs[b], idxs[b], fis[b]

        def body(g, carry):
            flat = row_off + g * (16 * K) + idx_v[pl.ds(g * 16, 16)]
            fi[pl.ds(g * 16, 16)] = flat
            plsc.store_scatter(buf, [flat], ovec)
            return carry

        lax.fori_loop(0, GROUPS, body, 0, unroll=5)

    def _clear(b):
        buf, fi = bufs[b], fis[b]

        def body(g, carry):
            plsc.store_scatter(buf, [fi[pl.ds(g * 16, 16)]], zvec)
            return carry

        lax.fori_loop(0, GROUPS, body, 0, unroll=5)

    def _stream_first(m, b):
        pltpu.make_async_copy(atom_hbm.at[pl.ds(0, CH)], idxs[b],
                              idxsems[b]).wait()
        _fill(b)
        chunk = wid + m * NW
        pltpu.async_copy(bufs[b], out_hbm.at[pl.ds(chunk * CHK, CHK)],
                         outsems[b])
        _prefetch_idx(m + 3, idxs[b], idxsems[b])

    def _stream_steady(m, b):
        pltpu.make_async_copy(bufs[b], out_hbm.at[pl.ds(0, CHK)],
                              outsems[b]).wait()
        _clear(b)
        pltpu.make_async_copy(atom_hbm.at[pl.ds(0, CH)], idxs[b],
                              idxsems[b]).wait()
        _fill(b)
        chunk = wid + m * NW
        pltpu.async_copy(bufs[b], out_hbm.at[pl.ds(chunk * CHK, CHK)],
                         outsems[b])

        @pl.when(m + 3 < n_mine)
        def _():
            _prefetch_idx(m + 3, idxs[b], idxsems[b])

    def _wait_scatters(t):
        pltpu.make_async_copy(onesA, out_hbm.at[fas[t]], ssems[t]).wait()
        pltpu.make_async_copy(onesB, out_hbm.at[fbs[t]], ssems[t]).wait()

    def _spmem_a(m, t, first):
        if not first:
            _wait_scatters(t)
        chunk = wid + m * NW
        base = chunk * CHK
        pltpu.async_copy(zsh, out_hbm.at[pl.ds(base, CHK)], zsems[t])
        pltpu.make_async_copy(atom_hbm.at[pl.ds(0, CH)], sidx[t],
                              sidxsems[t]).wait()
        sx, fa, fb = sidx[t], fas[t], fbs[t]

        def body(g, carry):
            flat = row_off + (base + g * (16 * K)) + sx[pl.ds(g * 16, 16)]
            fa[pl.ds(g * 16, 16)] = flat
            return carry

        lax.fori_loop(0, 8, body, 0, unroll=4)
        for g in (8, 9):
            flat = row_off + (base + g * (16 * K)) + sx[pl.ds(g * 16, 16)]
            fb[pl.ds((g - 8) * 16, 16)] = flat

        @pl.when(m + 6 < n_mine)
        def _():
            _prefetch_idx(m + 6, sidx[t], sidxsems[t])

    def _spmem_b(t):
        pltpu.make_async_copy(zsh, out_hbm.at[pl.ds(0, CHK)],
                              zsems[t]).wait()
        pltpu.async_copy(onesA, out_hbm.at[fas[t]], ssems[t])
        pltpu.async_copy(onesB, out_hbm.at[fbs[t]], ssems[t])

    _prefetch_idx(0, idxs[0], idxsems[0])
    _prefetch_idx(1, idxs[1], idxsems[1])
    _prefetch_idx(2, sidx[0], sidxsems[0])
    _prefetch_idx(5, sidx[1], sidxsems[1])

    for b in range(2):
        buf = bufs[b]

        def _zero_body(i, carry):
            buf[pl.ds(i * 16, 16)] = zvec
            return carry

        lax.fori_loop(0, CHK // 16, _zero_body, 0, unroll=8)
        if b == 0:
            pltpu.sync_copy(buf0.at[pl.ds(0, SLICE)],
                            zsh.at[pl.ds(sid * SLICE, SLICE)])
    plsc.subcore_barrier()

    for i in range(8):
        onesA[pl.ds(i * 16, 16)] = ovec
    for i in range(2):
        onesB[pl.ds(i * 16, 16)] = ovec

    _spmem_a(2, 0, True)
    _stream_first(0, 0)
    _stream_first(1, 1)
    _spmem_b(0)

    _spmem_a(5, 1, True)
    _stream_steady(3, 0)
    _stream_steady(4, 1)
    _spmem_b(1)

    def _pair_body(u, carry):
        for p in range(2):
            t = p
            m0 = 6 * u + 3 * p

            @pl.when(m0 + 2 < n_mine)
            def _():
                _spmem_a(m0 + 2, t, False)

            @pl.when(m0 < n_mine)
            def _():
                _stream_steady(m0, 0)

            @pl.when(m0 + 1 < n_mine)
            def _():
                _stream_steady(m0 + 1, 1)

            @pl.when(m0 + 2 < n_mine)
            def _():
                _spmem_b(t)
        return carry

    lax.fori_loop(1, 4, _pair_body, 0)

    pltpu.make_async_copy(buf0, out_hbm.at[pl.ds(0, CHK)], os0).wait()
    pltpu.make_async_copy(buf1, out_hbm.at[pl.ds(0, CHK)], os1).wait()
    _wait_scatters(0)
    _wait_scatters(1)


def kernel(atom_type, pos):
    del pos
    out = _one_hot_sc(atom_type.astype(jnp.int32))
    return out.reshape(N, K)
